# baseline (speedup 1.0000x reference)
import jax
import jax.numpy as jnp
from jax import lax
from jax.experimental import pallas as pl
from jax.experimental.pallas import tpu as pltpu
from jax.experimental.pallas import tpu_sc as plsc

HIDDEN = 768
EXPERTS = 64
EPS = 1e-06
TOKENS = 32768
BLOCK_T = 4096

NUM_WORKERS = 32
TOK_PER_WORKER = TOKENS // NUM_WORKERS
CHUNK = 512
CHUNKS_PER_WORKER = TOK_PER_WORKER // CHUNK
GROUPS_PER_CHUNK = CHUNK // 16


def _tc_logits_block(x_ref, wt_ref, b_ref, lg_ref):
    lg_ref[...] = (
        jnp.dot(x_ref[...], wt_ref[...], preferred_element_type=jnp.float32)
        + b_ref[...]
    )


def _tc_logits(x, wt, b2):
    return pl.pallas_call(
        _tc_logits_block,
        grid=(TOKENS // BLOCK_T,),
        in_specs=[
            pl.BlockSpec((BLOCK_T, HIDDEN), lambda i: (i, 0)),
            pl.BlockSpec((HIDDEN, EXPERTS), lambda i: (0, 0)),
            pl.BlockSpec((1, EXPERTS), lambda i: (0, 0)),
        ],
        out_specs=pl.BlockSpec((BLOCK_T, EXPERTS), lambda i: (i, 0)),
        out_shape=jax.ShapeDtypeStruct((TOKENS, EXPERTS), jnp.float32),
    )(x, wt, b2)


def _sc_route_body(lg_hbm, w_hbm, i_hbm, m_hbm, lg_c, m_c, w_c, i_c):
    wid = lax.axis_index("s") * 2 + lax.axis_index("c")
    iota = lax.broadcasted_iota(jnp.int32, (16,), 0)
    zeros16 = jnp.zeros((16,), jnp.float32)
    ones16 = jnp.ones((16,), jnp.float32)
    neginf = jnp.full((16,), -jnp.inf, jnp.float32)
    col0 = jnp.zeros((16,), jnp.int32)
    col1 = jnp.ones((16,), jnp.int32)

    def chunk_body(c, carry):
        tok_base = wid * TOK_PER_WORKER + c * CHUNK
        pltpu.sync_copy(lg_hbm.at[pl.ds(tok_base, CHUNK), :], lg_c)

        def group_body(g, gcarry):
            rows = [g * 64 + k * 16 + iota for k in range(4)]
            m1 = [neginf] * 4
            m2 = [neginf] * 4
            i1 = [jnp.zeros((16,), jnp.int32)] * 4
            i2 = [jnp.zeros((16,), jnp.int32)] * 4
            ssum = [zeros16] * 4
            for e in range(EXPERTS):
                e_vec = jnp.full((16,), e, jnp.int32)
                for k in range(4):
                    v = plsc.load_gather(lg_c, [rows[k], e_vec])
                    gt1 = v > m1[k]
                    gt2 = v > m2[k]
                    m2[k] = jnp.where(gt1, m1[k], jnp.where(gt2, v, m2[k]))
                    i2[k] = jnp.where(gt1, i1[k], jnp.where(gt2, e_vec, i2[k]))
                    m1[k] = jnp.where(gt1, v, m1[k])
                    i1[k] = jnp.where(gt1, e_vec, i1[k])
                    ssum[k] = ssum[k] + jnp.exp(v)
            for k in range(4):
                e1 = jnp.exp(m1[k])
                e2 = jnp.exp(m2[k])
                inv = 1.0 / (e1 + e2 + EPS * ssum[k])
                w1 = e1 * inv
                w2 = e2 * inv
                for j in range(EXPERTS):
                    plsc.store_scatter(
                        m_c, [rows[k], jnp.full((16,), j, jnp.int32)], zeros16
                    )
                plsc.store_scatter(m_c, [rows[k], i1[k]], ones16)
                plsc.store_scatter(m_c, [rows[k], i2[k]], ones16)
                plsc.store_scatter(w_c, [rows[k], col0], w1)
                plsc.store_scatter(w_c, [rows[k], col1], w2)
                plsc.store_scatter(i_c, [rows[k], col0], i1[k])
                plsc.store_scatter(i_c, [rows[k], col1], i2[k])
            return gcarry

        lax.fori_loop(0, GROUPS_PER_CHUNK // 4, group_body, 0, unroll=False)
        pltpu.sync_copy(w_c, w_hbm.at[pl.ds(tok_base, CHUNK), :])
        pltpu.sync_copy(i_c, i_hbm.at[pl.ds(tok_base, CHUNK), :])
        pltpu.sync_copy(m_c, m_hbm.at[pl.ds(tok_base, CHUNK), :])
        return carry

    lax.fori_loop(0, CHUNKS_PER_WORKER, chunk_body, 0, unroll=False)


def _sc_route(lg):
    mesh = plsc.VectorSubcoreMesh(core_axis_name="c", subcore_axis_name="s")
    fn = pl.kernel(
        _sc_route_body,
        out_type=[
            jax.ShapeDtypeStruct((TOKENS, 2), jnp.float32),
            jax.ShapeDtypeStruct((TOKENS, 2), jnp.int32),
            jax.ShapeDtypeStruct((TOKENS, EXPERTS), jnp.float32),
        ],
        mesh=mesh,
        scratch_types=[
            pltpu.VMEM((CHUNK, EXPERTS), jnp.float32),
            pltpu.VMEM((CHUNK, EXPERTS), jnp.float32),
            pltpu.VMEM((CHUNK, 2), jnp.float32),
            pltpu.VMEM((CHUNK, 2), jnp.int32),
        ],
        compiler_params=pltpu.CompilerParams(
            needs_layout_passes=False, use_tc_tiling_on_sc=False
        ),
    )
    return fn(lg)


def kernel(x, W, b):
    wt = W.T
    b2 = b.reshape(1, EXPERTS)
    lg = _tc_logits(x, wt, b2)
    w_o, i_o, m_o = _sc_route(lg)
    return (lg, w_o, i_o, m_o)


# speedup vs baseline: 1.0133x; 1.0133x over previous
import jax
import jax.numpy as jnp
from jax import lax
from jax.experimental import pallas as pl
from jax.experimental.pallas import tpu as pltpu
from jax.experimental.pallas import tpu_sc as plsc

HIDDEN = 768
EXPERTS = 64
EPS = 1e-06
TOKENS = 32768
BLOCK_T = 4096

NUM_WORKERS = 32
TOK_PER_WORKER = TOKENS // NUM_WORKERS
CHUNK = 512
CHUNKS_PER_WORKER = TOK_PER_WORKER // CHUNK
GROUPS_PER_CHUNK = CHUNK // 16


def _tc_logits_block(x_ref, wt_ref, b_ref, lg_ref):
    lg_ref[...] = (
        jnp.dot(x_ref[...], wt_ref[...], preferred_element_type=jnp.float32)
        + b_ref[...]
    )


def _tc_logits(x, wt, b2):
    return pl.pallas_call(
        _tc_logits_block,
        grid=(TOKENS // BLOCK_T,),
        in_specs=[
            pl.BlockSpec((BLOCK_T, HIDDEN), lambda i: (i, 0)),
            pl.BlockSpec((HIDDEN, EXPERTS), lambda i: (0, 0)),
            pl.BlockSpec((1, EXPERTS), lambda i: (0, 0)),
        ],
        out_specs=pl.BlockSpec((BLOCK_T, EXPERTS), lambda i: (i, 0)),
        out_shape=jax.ShapeDtypeStruct((TOKENS, EXPERTS), jnp.float32),
    )(x, wt, b2)


def _sc_route_body(lg_hbm, w_hbm, i_hbm, m_hbm, lg_c, m_c, w_c, i_c):
    wid = lax.axis_index("s") * 2 + lax.axis_index("c")
    iota = lax.broadcasted_iota(jnp.int32, (16,), 0)
    zeros16 = jnp.zeros((16,), jnp.float32)
    ones16 = jnp.ones((16,), jnp.float32)
    neginf = jnp.full((16,), -jnp.inf, jnp.float32)
    col0 = jnp.zeros((16,), jnp.int32)
    col1 = jnp.ones((16,), jnp.int32)

    def chunk_body(c, carry):
        tok_base = wid * TOK_PER_WORKER + c * CHUNK
        pltpu.sync_copy(lg_hbm.at[pl.ds(tok_base, CHUNK), :], lg_c)

        def group_body(g, gcarry):
            rows = [g * 64 + k * 16 + iota for k in range(4)]
            m1 = [neginf] * 4
            m2 = [neginf] * 4
            i1 = [jnp.zeros((16,), jnp.int32)] * 4
            i2 = [jnp.zeros((16,), jnp.int32)] * 4
            for e in range(EXPERTS):
                e_vec = jnp.full((16,), e, jnp.int32)
                for k in range(4):
                    v = plsc.load_gather(lg_c, [rows[k], e_vec])
                    gt1 = v > m1[k]
                    gt2 = v > m2[k]
                    m2[k] = jnp.where(gt1, m1[k], jnp.where(gt2, v, m2[k]))
                    i2[k] = jnp.where(gt1, i1[k], jnp.where(gt2, e_vec, i2[k]))
                    m1[k] = jnp.where(gt1, v, m1[k])
                    i1[k] = jnp.where(gt1, e_vec, i1[k])
            for k in range(4):
                e2 = jnp.exp(m2[k] - m1[k])
                inv = 1.0 / (1.0 + e2)
                w1 = inv
                w2 = e2 * inv
                for j in range(EXPERTS):
                    plsc.store_scatter(
                        m_c, [rows[k], jnp.full((16,), j, jnp.int32)], zeros16
                    )
                plsc.store_scatter(m_c, [rows[k], i1[k]], ones16)
                plsc.store_scatter(m_c, [rows[k], i2[k]], ones16)
                plsc.store_scatter(w_c, [rows[k], col0], w1)
                plsc.store_scatter(w_c, [rows[k], col1], w2)
                plsc.store_scatter(i_c, [rows[k], col0], i1[k])
                plsc.store_scatter(i_c, [rows[k], col1], i2[k])
            return gcarry

        lax.fori_loop(0, GROUPS_PER_CHUNK // 4, group_body, 0, unroll=False)
        pltpu.sync_copy(w_c, w_hbm.at[pl.ds(tok_base, CHUNK), :])
        pltpu.sync_copy(i_c, i_hbm.at[pl.ds(tok_base, CHUNK), :])
        pltpu.sync_copy(m_c, m_hbm.at[pl.ds(tok_base, CHUNK), :])
        return carry

    lax.fori_loop(0, CHUNKS_PER_WORKER, chunk_body, 0, unroll=False)


def _sc_route(lg):
    mesh = plsc.VectorSubcoreMesh(core_axis_name="c", subcore_axis_name="s")
    fn = pl.kernel(
        _sc_route_body,
        out_type=[
            jax.ShapeDtypeStruct((TOKENS, 2), jnp.float32),
            jax.ShapeDtypeStruct((TOKENS, 2), jnp.int32),
            jax.ShapeDtypeStruct((TOKENS, EXPERTS), jnp.float32),
        ],
        mesh=mesh,
        scratch_types=[
            pltpu.VMEM((CHUNK, EXPERTS), jnp.float32),
            pltpu.VMEM((CHUNK, EXPERTS), jnp.float32),
            pltpu.VMEM((CHUNK, 2), jnp.float32),
            pltpu.VMEM((CHUNK, 2), jnp.int32),
        ],
        compiler_params=pltpu.CompilerParams(
            needs_layout_passes=False, use_tc_tiling_on_sc=False
        ),
    )
    return fn(lg)


def kernel(x, W, b):
    wt = W.T
    b2 = b.reshape(1, EXPERTS)
    lg = _tc_logits(x, wt, b2)
    w_o, i_o, m_o = _sc_route(lg)
    return (lg, w_o, i_o, m_o)


# speedup vs baseline: 1.3145x; 1.2972x over previous
import jax
import jax.numpy as jnp
from jax import lax
from jax.experimental import pallas as pl
from jax.experimental.pallas import tpu as pltpu
from jax.experimental.pallas import tpu_sc as plsc

HIDDEN = 768
EXPERTS = 64
EPS = 1e-06
TOKENS = 32768
BLOCK_T = 4096

NUM_WORKERS = 32
TOK_PER_WORKER = TOKENS // NUM_WORKERS
CHUNK = 512
CHUNKS_PER_WORKER = TOK_PER_WORKER // CHUNK
GROUPS_PER_CHUNK = CHUNK // 64


def _tc_logits_block(x_ref, wt_ref, b_ref, lg_ref, lgt_ref):
    xb = x_ref[...]
    wt = wt_ref[...]
    b = b_ref[...]
    lg_ref[...] = (
        jnp.dot(xb, wt, preferred_element_type=jnp.float32) + b
    )
    lgt = lax.dot_general(
        wt, xb, (((0,), (1,)), ((), ())), preferred_element_type=jnp.float32
    )
    lgt_ref[...] = lgt + b.reshape(EXPERTS, 1)


def _tc_logits(x, wt, b2):
    return pl.pallas_call(
        _tc_logits_block,
        grid=(TOKENS // BLOCK_T,),
        in_specs=[
            pl.BlockSpec((BLOCK_T, HIDDEN), lambda i: (i, 0)),
            pl.BlockSpec((HIDDEN, EXPERTS), lambda i: (0, 0)),
            pl.BlockSpec((1, EXPERTS), lambda i: (0, 0)),
        ],
        out_specs=[
            pl.BlockSpec((BLOCK_T, EXPERTS), lambda i: (i, 0)),
            pl.BlockSpec((EXPERTS, BLOCK_T), lambda i: (0, i)),
        ],
        out_shape=[
            jax.ShapeDtypeStruct((TOKENS, EXPERTS), jnp.float32),
            jax.ShapeDtypeStruct((EXPERTS, TOKENS), jnp.float32),
        ],
    )(x, wt, b2)


def _sc_route_body(lgt_hbm, w_hbm, i_hbm, m_hbm, lgt_c, m_c, w_c, i_c):
    wid = lax.axis_index("s") * 2 + lax.axis_index("c")
    iota = lax.broadcasted_iota(jnp.int32, (16,), 0)
    zeros16 = jnp.zeros((16,), jnp.float32)
    ones16 = jnp.ones((16,), jnp.float32)
    neginf = jnp.full((16,), -jnp.inf, jnp.float32)
    col0 = jnp.zeros((16,), jnp.int32)
    col1 = jnp.ones((16,), jnp.int32)

    def chunk_body(c, carry):
        tok_base = wid * TOK_PER_WORKER + c * CHUNK
        pltpu.sync_copy(lgt_hbm.at[:, pl.ds(tok_base, CHUNK)], lgt_c)

        def group_body(g, gcarry):
            base = g * 64
            rows = [base + k * 16 + iota for k in range(4)]
            m1 = [neginf] * 4
            m2 = [neginf] * 4
            i1 = [jnp.zeros((16,), jnp.int32)] * 4
            i2 = [jnp.zeros((16,), jnp.int32)] * 4
            for e in range(EXPERTS):
                e_vec = jnp.full((16,), e, jnp.int32)
                for k in range(4):
                    v = lgt_c[e, pl.ds(base + k * 16, 16)]
                    gt1 = v > m1[k]
                    gt2 = v > m2[k]
                    m2[k] = jnp.where(gt1, m1[k], jnp.where(gt2, v, m2[k]))
                    i2[k] = jnp.where(gt1, i1[k], jnp.where(gt2, e_vec, i2[k]))
                    m1[k] = jnp.where(gt1, v, m1[k])
                    i1[k] = jnp.where(gt1, e_vec, i1[k])
            for k in range(4):
                e2 = jnp.exp(m2[k] - m1[k])
                inv = 1.0 / (1.0 + e2)
                w1 = inv
                w2 = e2 * inv
                for t in range(16):
                    for j in range(EXPERTS // 16):
                        m_c[base + k * 16 + t, pl.ds(j * 16, 16)] = zeros16
                plsc.store_scatter(m_c, [rows[k], i1[k]], ones16)
                plsc.store_scatter(m_c, [rows[k], i2[k]], ones16)
                plsc.store_scatter(w_c, [rows[k], col0], w1)
                plsc.store_scatter(w_c, [rows[k], col1], w2)
                plsc.store_scatter(i_c, [rows[k], col0], i1[k])
                plsc.store_scatter(i_c, [rows[k], col1], i2[k])
            return gcarry

        lax.fori_loop(0, GROUPS_PER_CHUNK, group_body, 0, unroll=False)
        pltpu.sync_copy(w_c, w_hbm.at[pl.ds(tok_base, CHUNK), :])
        pltpu.sync_copy(i_c, i_hbm.at[pl.ds(tok_base, CHUNK), :])
        pltpu.sync_copy(m_c, m_hbm.at[pl.ds(tok_base, CHUNK), :])
        return carry

    lax.fori_loop(0, CHUNKS_PER_WORKER, chunk_body, 0, unroll=False)


def _sc_route(lgt):
    mesh = plsc.VectorSubcoreMesh(core_axis_name="c", subcore_axis_name="s")
    fn = pl.kernel(
        _sc_route_body,
        out_type=[
            jax.ShapeDtypeStruct((TOKENS, 2), jnp.float32),
            jax.ShapeDtypeStruct((TOKENS, 2), jnp.int32),
            jax.ShapeDtypeStruct((TOKENS, EXPERTS), jnp.float32),
        ],
        mesh=mesh,
        scratch_types=[
            pltpu.VMEM((EXPERTS, CHUNK), jnp.float32),
            pltpu.VMEM((CHUNK, EXPERTS), jnp.float32),
            pltpu.VMEM((CHUNK, 2), jnp.float32),
            pltpu.VMEM((CHUNK, 2), jnp.int32),
        ],
        compiler_params=pltpu.CompilerParams(
            needs_layout_passes=False, use_tc_tiling_on_sc=False
        ),
    )
    return fn(lgt)


def kernel(x, W, b):
    wt = W.T
    b2 = b.reshape(1, EXPERTS)
    lg, lgt = _tc_logits(x, wt, b2)
    w_o, i_o, m_o = _sc_route(lgt)
    return (lg, w_o, i_o, m_o)


# speedup vs baseline: 1.6355x; 1.2442x over previous
import jax
import jax.numpy as jnp
from jax import lax
from jax.experimental import pallas as pl
from jax.experimental.pallas import tpu as pltpu
from jax.experimental.pallas import tpu_sc as plsc

HIDDEN = 768
EXPERTS = 64
EPS = 1e-06
TOKENS = 32768
BLOCK_T = 4096

NUM_WORKERS = 32
TOK_PER_WORKER = TOKENS // NUM_WORKERS
CHUNK = 256
CHUNKS_PER_WORKER = TOK_PER_WORKER // CHUNK
GROUPS_PER_CHUNK = CHUNK // 64


def _tc_logits_block(x_ref, wt_ref, b_ref, lg_ref, lgt_ref):
    xb = x_ref[...]
    wt = wt_ref[...]
    b = b_ref[...]
    lg_ref[...] = (
        jnp.dot(xb, wt, preferred_element_type=jnp.float32) + b
    )
    lgt = lax.dot_general(
        wt, xb, (((0,), (1,)), ((), ())), preferred_element_type=jnp.float32
    )
    lgt_ref[...] = lgt + b.reshape(EXPERTS, 1)


def _tc_logits(x, wt, b2):
    return pl.pallas_call(
        _tc_logits_block,
        grid=(TOKENS // BLOCK_T,),
        in_specs=[
            pl.BlockSpec((BLOCK_T, HIDDEN), lambda i: (i, 0)),
            pl.BlockSpec((HIDDEN, EXPERTS), lambda i: (0, 0)),
            pl.BlockSpec((1, EXPERTS), lambda i: (0, 0)),
        ],
        out_specs=[
            pl.BlockSpec((BLOCK_T, EXPERTS), lambda i: (i, 0)),
            pl.BlockSpec((EXPERTS, BLOCK_T), lambda i: (0, i)),
        ],
        out_shape=[
            jax.ShapeDtypeStruct((TOKENS, EXPERTS), jnp.float32),
            jax.ShapeDtypeStruct((EXPERTS, TOKENS), jnp.float32),
        ],
    )(x, wt, b2)


def _sc_route_body(lgt_hbm, w_hbm, i_hbm, m_hbm, lgt_c, m_c, w_c, i_c):
    wid = lax.axis_index("s") * 2 + lax.axis_index("c")
    iota = lax.broadcasted_iota(jnp.int32, (16,), 0)
    zeros16 = jnp.zeros((16,), jnp.float32)
    ones16 = jnp.ones((16,), jnp.float32)
    neginf = jnp.full((16,), -jnp.inf, jnp.float32)
    col0 = jnp.zeros((16,), jnp.int32)
    col1 = jnp.ones((16,), jnp.int32)

    def chunk_body(c, carry):
        tok_base = wid * TOK_PER_WORKER + c * CHUNK
        pltpu.sync_copy(lgt_hbm.at[:, pl.ds(tok_base, CHUNK)], lgt_c)

        def group_body(g, gcarry):
            base = g * 64
            rows = [base + k * 16 + iota for k in range(4)]
            m1 = [neginf] * 4
            m2 = [neginf] * 4
            i1 = [jnp.zeros((16,), jnp.int32)] * 4
            i2 = [jnp.zeros((16,), jnp.int32)] * 4
            for e in range(EXPERTS):
                e_vec = jnp.full((16,), e, jnp.int32)
                for k in range(4):
                    v = lgt_c[e, pl.ds(base + k * 16, 16)]
                    gt1 = v > m1[k]
                    gt2 = v > m2[k]
                    m2[k] = jnp.where(gt1, m1[k], jnp.where(gt2, v, m2[k]))
                    i2[k] = jnp.where(gt1, i1[k], jnp.where(gt2, e_vec, i2[k]))
                    m1[k] = jnp.where(gt1, v, m1[k])
                    i1[k] = jnp.where(gt1, e_vec, i1[k])
            for k in range(4):
                e2 = jnp.exp(m2[k] - m1[k])
                inv = 1.0 / (1.0 + e2)
                w1 = inv
                w2 = e2 * inv
                for t in range(16):
                    for j in range(EXPERTS // 16):
                        m_c[base + k * 16 + t, pl.ds(j * 16, 16)] = zeros16
                plsc.store_scatter(m_c, [rows[k], i1[k]], ones16)
                plsc.store_scatter(m_c, [rows[k], i2[k]], ones16)
                plsc.store_scatter(w_c, [rows[k], col0], w1)
                plsc.store_scatter(w_c, [rows[k], col1], w2)
                plsc.store_scatter(i_c, [rows[k], col0], i1[k])
                plsc.store_scatter(i_c, [rows[k], col1], i2[k])
            return gcarry

        lax.fori_loop(0, GROUPS_PER_CHUNK, group_body, 0, unroll=False)
        pltpu.sync_copy(w_c, w_hbm.at[pl.ds(tok_base, CHUNK), :])
        pltpu.sync_copy(i_c, i_hbm.at[pl.ds(tok_base, CHUNK), :])
        pltpu.sync_copy(m_c, m_hbm.at[pl.ds(tok_base, CHUNK), :])
        return carry

    lax.fori_loop(0, CHUNKS_PER_WORKER, chunk_body, 0, unroll=False)


def _sc_route(lgt):
    mesh = plsc.VectorSubcoreMesh(core_axis_name="c", subcore_axis_name="s")
    fn = pl.kernel(
        _sc_route_body,
        out_type=[
            jax.ShapeDtypeStruct((TOKENS, 2), jnp.float32),
            jax.ShapeDtypeStruct((TOKENS, 2), jnp.int32),
            jax.ShapeDtypeStruct((TOKENS, EXPERTS), jnp.float32),
        ],
        mesh=mesh,
        scratch_types=[
            pltpu.VMEM((EXPERTS, CHUNK), jnp.float32),
            pltpu.VMEM((CHUNK, EXPERTS), jnp.float32),
            pltpu.VMEM((CHUNK, 2), jnp.float32),
            pltpu.VMEM((CHUNK, 2), jnp.int32),
        ],
        compiler_params=pltpu.CompilerParams(
            needs_layout_passes=False, use_tc_tiling_on_sc=True
        ),
    )
    return fn(lgt)


def kernel(x, W, b):
    wt = W.T
    b2 = b.reshape(1, EXPERTS)
    lg, lgt = _tc_logits(x, wt, b2)
    w_o, i_o, m_o = _sc_route(lgt)
    return (lg, w_o, i_o, m_o)


# speedup vs baseline: 2.1917x; 1.3401x over previous
import jax
import jax.numpy as jnp
from jax import lax
from jax.experimental import pallas as pl
from jax.experimental.pallas import tpu as pltpu
from jax.experimental.pallas import tpu_sc as plsc

HIDDEN = 768
EXPERTS = 64
EPS = 1e-06
TOKENS = 32768
BLOCK_T = 4096

NUM_WORKERS = 32
TOK_PER_WORKER = TOKENS // NUM_WORKERS
CHUNK = 512
CHUNKS_PER_WORKER = TOK_PER_WORKER // CHUNK
GROUPS_PER_CHUNK = CHUNK // 64


def _tc_logits_block(x_ref, wt_ref, b_ref, lg_ref, lgt_ref):
    xb = x_ref[...]
    wt = wt_ref[...]
    b = b_ref[...]
    lg_ref[...] = (
        jnp.dot(xb, wt, preferred_element_type=jnp.float32) + b
    )
    lgt = lax.dot_general(
        wt, xb, (((0,), (1,)), ((), ())), preferred_element_type=jnp.float32
    )
    lgt_ref[...] = lgt + b.reshape(EXPERTS, 1)


def _tc_logits(x, wt, b2):
    return pl.pallas_call(
        _tc_logits_block,
        grid=(TOKENS // BLOCK_T,),
        in_specs=[
            pl.BlockSpec((BLOCK_T, HIDDEN), lambda i: (i, 0)),
            pl.BlockSpec((HIDDEN, EXPERTS), lambda i: (0, 0)),
            pl.BlockSpec((1, EXPERTS), lambda i: (0, 0)),
        ],
        out_specs=[
            pl.BlockSpec((BLOCK_T, EXPERTS), lambda i: (i, 0)),
            pl.BlockSpec((EXPERTS, BLOCK_T), lambda i: (0, i)),
        ],
        out_shape=[
            jax.ShapeDtypeStruct((TOKENS, EXPERTS), jnp.float32),
            jax.ShapeDtypeStruct((EXPERTS, TOKENS), jnp.float32),
        ],
    )(x, wt, b2)


def _sc_route_body(lgt_hbm, w_hbm, i_hbm, m_hbm, lgt_c, m_c, w_c, i_c):
    wid = lax.axis_index("s") * 2 + lax.axis_index("c")
    iota = lax.broadcasted_iota(jnp.int32, (16,), 0)
    zeros16 = jnp.zeros((16,), jnp.float32)
    ones16 = jnp.ones((16,), jnp.float32)
    neginf = jnp.full((16,), -jnp.inf, jnp.float32)
    col0 = jnp.zeros((16,), jnp.int32)
    col1 = jnp.ones((16,), jnp.int32)

    def chunk_body(c, carry):
        tok_base = wid * TOK_PER_WORKER + c * CHUNK
        pltpu.sync_copy(lgt_hbm.at[:, pl.ds(tok_base, CHUNK)], lgt_c)

        def group_body(g, gcarry):
            base = g * 64
            rows = [base + k * 16 + iota for k in range(4)]
            m1 = [neginf] * 4
            m2 = [neginf] * 4
            i1 = [jnp.zeros((16,), jnp.int32)] * 4
            i2 = [jnp.zeros((16,), jnp.int32)] * 4
            for e in range(EXPERTS):
                e_vec = jnp.full((16,), e, jnp.int32)
                for k in range(4):
                    v = lgt_c[e, pl.ds(base + k * 16, 16)]
                    gt1 = v > m1[k]
                    gt2 = v > m2[k]
                    m2[k] = jnp.where(gt1, m1[k], jnp.where(gt2, v, m2[k]))
                    i2[k] = jnp.where(gt1, i1[k], jnp.where(gt2, e_vec, i2[k]))
                    m1[k] = jnp.where(gt1, v, m1[k])
                    i1[k] = jnp.where(gt1, e_vec, i1[k])
            for k in range(4):
                e2 = jnp.exp(m2[k] - m1[k])
                inv = 1.0 / (1.0 + e2)
                w1 = inv
                w2 = e2 * inv
                for t in range(16):
                    for j in range(EXPERTS // 16):
                        m_c[base + k * 16 + t, pl.ds(j * 16, 16)] = zeros16
                plsc.store_scatter(m_c, [rows[k], i1[k]], ones16)
                plsc.store_scatter(m_c, [rows[k], i2[k]], ones16)
                w_c[0, pl.ds(base + k * 16, 16)] = w1
                w_c[1, pl.ds(base + k * 16, 16)] = w2
                i_c[0, pl.ds(base + k * 16, 16)] = i1[k]
                i_c[1, pl.ds(base + k * 16, 16)] = i2[k]
            return gcarry

        lax.fori_loop(0, GROUPS_PER_CHUNK, group_body, 0, unroll=False)
        pltpu.sync_copy(w_c.at[pl.ds(0, 2), :], w_hbm.at[:, pl.ds(tok_base, CHUNK)])
        pltpu.sync_copy(i_c.at[pl.ds(0, 2), :], i_hbm.at[:, pl.ds(tok_base, CHUNK)])
        pltpu.sync_copy(m_c, m_hbm.at[pl.ds(tok_base, CHUNK), :])
        return carry

    lax.fori_loop(0, CHUNKS_PER_WORKER, chunk_body, 0, unroll=False)


def _sc_route(lgt):
    mesh = plsc.VectorSubcoreMesh(core_axis_name="c", subcore_axis_name="s")
    fn = pl.kernel(
        _sc_route_body,
        out_type=[
            jax.ShapeDtypeStruct((2, TOKENS), jnp.float32),
            jax.ShapeDtypeStruct((2, TOKENS), jnp.int32),
            jax.ShapeDtypeStruct((TOKENS, EXPERTS), jnp.float32),
        ],
        mesh=mesh,
        scratch_types=[
            pltpu.VMEM((EXPERTS, CHUNK), jnp.float32),
            pltpu.VMEM((CHUNK, EXPERTS), jnp.float32),
            pltpu.VMEM((8, CHUNK), jnp.float32),
            pltpu.VMEM((8, CHUNK), jnp.int32),
        ],
        compiler_params=pltpu.CompilerParams(
            needs_layout_passes=False, use_tc_tiling_on_sc=True
        ),
    )
    return fn(lgt)


def kernel(x, W, b):
    wt = W.T
    b2 = b.reshape(1, EXPERTS)
    lg, lgt = _tc_logits(x, wt, b2)
    w_o, i_o, m_o = _sc_route(lgt)
    return (lg, w_o.T, i_o.T, m_o)


# speedup vs baseline: 2.3275x; 1.0620x over previous
import jax
import jax.numpy as jnp
from jax import lax
from jax.experimental import pallas as pl
from jax.experimental.pallas import tpu as pltpu
from jax.experimental.pallas import tpu_sc as plsc

HIDDEN = 768
EXPERTS = 64
EPS = 1e-06
TOKENS = 32768
BLOCK_T = 4096

NUM_WORKERS = 32
TOK_PER_WORKER = TOKENS // NUM_WORKERS
CHUNK = 256
CHUNKS_PER_WORKER = TOK_PER_WORKER // CHUNK
GROUPS_PER_CHUNK = CHUNK // 64


def _tc_logits_block(x_ref, wt_ref, b_ref, lg_ref, lgt_ref):
    xb = x_ref[...]
    wt = wt_ref[...]
    b = b_ref[...]
    lg_ref[...] = (
        jnp.dot(xb, wt, preferred_element_type=jnp.float32) + b
    )
    lgt = lax.dot_general(
        wt, xb, (((0,), (1,)), ((), ())), preferred_element_type=jnp.float32
    )
    lgt_ref[...] = lgt + b.reshape(EXPERTS, 1)


def _tc_logits(x, wt, b2):
    return pl.pallas_call(
        _tc_logits_block,
        grid=(TOKENS // BLOCK_T,),
        in_specs=[
            pl.BlockSpec((BLOCK_T, HIDDEN), lambda i: (i, 0)),
            pl.BlockSpec((HIDDEN, EXPERTS), lambda i: (0, 0)),
            pl.BlockSpec((1, EXPERTS), lambda i: (0, 0)),
        ],
        out_specs=[
            pl.BlockSpec((BLOCK_T, EXPERTS), lambda i: (i, 0)),
            pl.BlockSpec((EXPERTS, BLOCK_T), lambda i: (0, i)),
        ],
        out_shape=[
            jax.ShapeDtypeStruct((TOKENS, EXPERTS), jnp.float32),
            jax.ShapeDtypeStruct((EXPERTS, TOKENS), jnp.float32),
        ],
    )(x, wt, b2)


def _sc_route_body(
    lgt_hbm, w_hbm, i_hbm, m_hbm,
    lgt_c0, lgt_c1, m_c0, m_c1, w_c0, w_c1, i_c0, i_c1,
    sin0, sin1, sout0, sout1,
):
    wid = lax.axis_index("s") * 2 + lax.axis_index("c")
    iota = lax.broadcasted_iota(jnp.int32, (16,), 0)
    zeros16 = jnp.zeros((16,), jnp.float32)
    ones16 = jnp.ones((16,), jnp.float32)
    neginf = jnp.full((16,), -jnp.inf, jnp.float32)
    bufs = (
        (lgt_c0, m_c0, w_c0, i_c0, sin0, sout0),
        (lgt_c1, m_c1, w_c1, i_c1, sin1, sout1),
    )

    def in_start(c, lgt_b, sin_b):
        off = wid * TOK_PER_WORKER + c * CHUNK
        pltpu.async_copy(lgt_hbm.at[:, pl.ds(off, CHUNK)], lgt_b, sin_b)

    def in_wait(lgt_b, sin_b):
        pltpu.make_async_copy(
            lgt_hbm.at[:, pl.ds(0, CHUNK)], lgt_b, sin_b
        ).wait()

    def out_start(tok_base, m_b, w_b, i_b, sout_b):
        pltpu.async_copy(m_b, m_hbm.at[pl.ds(tok_base, CHUNK), :], sout_b)
        pltpu.async_copy(
            w_b.at[pl.ds(0, 2), :], w_hbm.at[:, pl.ds(tok_base, CHUNK)], sout_b
        )
        pltpu.async_copy(
            i_b.at[pl.ds(0, 2), :], i_hbm.at[:, pl.ds(tok_base, CHUNK)], sout_b
        )

    def out_wait(m_b, w_b, i_b, sout_b):
        pltpu.make_async_copy(
            m_b, m_hbm.at[pl.ds(0, CHUNK), :], sout_b
        ).wait()
        pltpu.make_async_copy(
            w_b.at[pl.ds(0, 2), :], w_hbm.at[:, pl.ds(0, CHUNK)], sout_b
        ).wait()
        pltpu.make_async_copy(
            i_b.at[pl.ds(0, 2), :], i_hbm.at[:, pl.ds(0, CHUNK)], sout_b
        ).wait()

    def compute_chunk(lgt_b, m_b, w_b, i_b):
        def group_body(g, gcarry):
            base = g * 64
            rows = [base + k * 16 + iota for k in range(4)]
            m1 = [neginf] * 4
            m2 = [neginf] * 4
            i1 = [jnp.zeros((16,), jnp.int32)] * 4
            i2 = [jnp.zeros((16,), jnp.int32)] * 4
            for e in range(EXPERTS):
                e_vec = jnp.full((16,), e, jnp.int32)
                for k in range(4):
                    v = lgt_b[e, pl.ds(base + k * 16, 16)]
                    gt1 = v > m1[k]
                    gt2 = v > m2[k]
                    m2[k] = jnp.where(gt1, m1[k], jnp.where(gt2, v, m2[k]))
                    i2[k] = jnp.where(gt1, i1[k], jnp.where(gt2, e_vec, i2[k]))
                    m1[k] = jnp.where(gt1, v, m1[k])
                    i1[k] = jnp.where(gt1, e_vec, i1[k])
            for k in range(4):
                e2 = jnp.exp(m2[k] - m1[k])
                inv = 1.0 / (1.0 + e2)
                w1 = inv
                w2 = e2 * inv
                for t in range(16):
                    for j in range(EXPERTS // 16):
                        m_b[base + k * 16 + t, pl.ds(j * 16, 16)] = zeros16
                plsc.store_scatter(m_b, [rows[k], i1[k]], ones16)
                plsc.store_scatter(m_b, [rows[k], i2[k]], ones16)
                w_b[0, pl.ds(base + k * 16, 16)] = w1
                w_b[1, pl.ds(base + k * 16, 16)] = w2
                i_b[0, pl.ds(base + k * 16, 16)] = i1[k]
                i_b[1, pl.ds(base + k * 16, 16)] = i2[k]
            return gcarry

        lax.fori_loop(0, GROUPS_PER_CHUNK, group_body, 0, unroll=False)

    in_start(0, lgt_c0, sin0)

    def cc_body(cc, carry):
        for b in range(2):
            lgt_b, m_b, w_b, i_b, sin_b, sout_b = bufs[b]
            c = 2 * cc + b
            tok_base = wid * TOK_PER_WORKER + c * CHUNK
            in_wait(lgt_b, sin_b)
            nlgt_b, _, _, _, nsin_b, _ = bufs[1 - b]
            if b == 0:
                in_start(c + 1, nlgt_b, nsin_b)
            else:
                @pl.when(cc < (CHUNKS_PER_WORKER // 2) - 1)
                def _():
                    in_start(c + 1, nlgt_b, nsin_b)

            @pl.when(cc >= 1)
            def _():
                out_wait(m_b, w_b, i_b, sout_b)

            compute_chunk(lgt_b, m_b, w_b, i_b)
            out_start(tok_base, m_b, w_b, i_b, sout_b)
        return carry

    lax.fori_loop(0, CHUNKS_PER_WORKER // 2, cc_body, 0, unroll=False)
    for b in range(2):
        lgt_b, m_b, w_b, i_b, sin_b, sout_b = bufs[b]
        out_wait(m_b, w_b, i_b, sout_b)


def _sc_route(lgt):
    mesh = plsc.VectorSubcoreMesh(core_axis_name="c", subcore_axis_name="s")
    fn = pl.kernel(
        _sc_route_body,
        out_type=[
            jax.ShapeDtypeStruct((2, TOKENS), jnp.float32),
            jax.ShapeDtypeStruct((2, TOKENS), jnp.int32),
            jax.ShapeDtypeStruct((TOKENS, EXPERTS), jnp.float32),
        ],
        mesh=mesh,
        scratch_types=[
            pltpu.VMEM((EXPERTS, CHUNK), jnp.float32),
            pltpu.VMEM((EXPERTS, CHUNK), jnp.float32),
            pltpu.VMEM((CHUNK, EXPERTS), jnp.float32),
            pltpu.VMEM((CHUNK, EXPERTS), jnp.float32),
            pltpu.VMEM((8, CHUNK), jnp.float32),
            pltpu.VMEM((8, CHUNK), jnp.float32),
            pltpu.VMEM((8, CHUNK), jnp.int32),
            pltpu.VMEM((8, CHUNK), jnp.int32),
            pltpu.SemaphoreType.DMA,
            pltpu.SemaphoreType.DMA,
            pltpu.SemaphoreType.DMA,
            pltpu.SemaphoreType.DMA,
        ],
        compiler_params=pltpu.CompilerParams(
            needs_layout_passes=False, use_tc_tiling_on_sc=True
        ),
    )
    return fn(lgt)


def kernel(x, W, b):
    wt = W.T
    b2 = b.reshape(1, EXPERTS)
    lg, lgt = _tc_logits(x, wt, b2)
    w_o, i_o, m_o = _sc_route(lgt)
    return (lg, w_o.T, i_o.T, m_o)
